# baseline (speedup 1.0000x reference)
import functools

import jax
import jax.numpy as jnp
from jax import lax
from jax.experimental import pallas as pl
from jax.experimental.pallas import tpu as pltpu
from jax.experimental.pallas import tpu_sc as plsc

VOCAB = 100000
SEQ = 2048
BATCH = 4
EMBED = 1024

NC = 2
NS = 16
NW = NC * NS

ROWS = BATCH * SEQ
SLAB = SEQ
RPW = SLAB // NW
CHUNK = 16
NCHUNK = RPW // CHUNK
NBUF = 4

_mesh = plsc.VectorSubcoreMesh(
    core_axis_name="c", subcore_axis_name="s", num_cores=NC, num_subcores=NS
)


@functools.partial(
    pl.kernel,
    out_type=jax.ShapeDtypeStruct((SLAB, EMBED), jnp.float32),
    mesh=_mesh,
    compiler_params=pltpu.CompilerParams(needs_layout_passes=False),
    scratch_types=[
        pltpu.VMEM((RPW,), jnp.int32),
        pltpu.VMEM((NBUF, CHUNK, EMBED), jnp.float32),
        [pltpu.SemaphoreType.DMA] * NBUF,
    ],
)
def _sc_gather(ids_hbm, table_hbm, out_hbm, idx_v, bufs, sems):
    wid = lax.axis_index("s") * NC + lax.axis_index("c")
    base = wid * RPW
    pltpu.sync_copy(ids_hbm.at[pl.ds(base, RPW)], idx_v)

    def start(k, b):
        pltpu.async_copy(
            table_hbm.at[idx_v.at[pl.ds(k * CHUNK, CHUNK)]], bufs.at[b], sems[b]
        )

    def wait(k, b):
        pltpu.make_async_copy(
            table_hbm.at[idx_v.at[pl.ds(k * CHUNK, CHUNK)]], bufs.at[b], sems[b]
        ).wait()

    for b in range(NBUF):
        start(b, b)
    for k in range(NCHUNK):
        b = k % NBUF
        wait(k, b)
        pltpu.sync_copy(bufs.at[b], out_hbm.at[pl.ds(base + k * CHUNK, CHUNK)])


BM = 512


def _ln_block(emb_ref, pos_ref, g_ref, b_ref, out_ref):
    e = emb_ref[...] + pos_ref[...]
    mean = jnp.mean(e, axis=-1, keepdims=True)
    var = jnp.mean(e * e, axis=-1, keepdims=True) - mean * mean
    out_ref[...] = (e - mean) * lax.rsqrt(var + 1e-5) * g_ref[...] + b_ref[...]


def _make_tc_ln(slab):
    if slab == 0:
        body = _ln_block
    else:
        def body(emb_ref, pos_ref, g_ref, b_ref, carry_ref, out_ref):
            del carry_ref
            _ln_block(emb_ref, pos_ref, g_ref, b_ref, out_ref)

    in_specs = [
        pl.BlockSpec((BM, EMBED), lambda i: (i, 0)),
        pl.BlockSpec((BM, EMBED), lambda i: (i, 0)),
        pl.BlockSpec((1, EMBED), lambda i: (0, 0)),
        pl.BlockSpec((1, EMBED), lambda i: (0, 0)),
    ]
    if slab > 0:
        in_specs.append(pl.BlockSpec(memory_space=pl.ANY))

    return pl.pallas_call(
        body,
        grid=(SEQ // BM,),
        in_specs=in_specs,
        out_specs=pl.BlockSpec(
            (BM, EMBED), lambda i, s=slab: (s * (SEQ // BM) + i, 0)
        ),
        out_shape=jax.ShapeDtypeStruct((ROWS, EMBED), jnp.float32),
        input_output_aliases={4: 0} if slab > 0 else {},
    )


_tc_ln = [_make_tc_ln(j) for j in range(BATCH)]


def kernel(input_ids, token_table, pos_table, gamma, beta):
    flat_ids = input_ids.reshape(-1).astype(jnp.int32)
    g2 = gamma.reshape(1, EMBED)
    b2 = beta.reshape(1, EMBED)
    embs = [
        _sc_gather(flat_ids[j * SLAB:(j + 1) * SLAB], token_table)
        for j in range(BATCH)
    ]
    carry = _tc_ln[0](embs[0], pos_table, g2, b2)
    for j in range(1, BATCH):
        carry = _tc_ln[j](embs[j], pos_table, g2, b2, carry)
    return carry.reshape(BATCH, SEQ, EMBED)


# speedup vs baseline: 1.1724x; 1.1724x over previous
import functools

import jax
import jax.numpy as jnp
from jax import lax
from jax.experimental import pallas as pl
from jax.experimental.pallas import tpu as pltpu
from jax.experimental.pallas import tpu_sc as plsc

VOCAB = 100000
SEQ = 2048
BATCH = 4
EMBED = 1024

NC = 2
NS = 16
NW = NC * NS

ROWS = BATCH * SEQ
RPW = ROWS // NW
CHUNK = 16
NCHUNK = RPW // CHUNK
NBUF = 4

_mesh = plsc.VectorSubcoreMesh(
    core_axis_name="c", subcore_axis_name="s", num_cores=NC, num_subcores=NS
)


@functools.partial(
    pl.kernel,
    out_type=jax.ShapeDtypeStruct((ROWS, EMBED), jnp.float32),
    mesh=_mesh,
    compiler_params=pltpu.CompilerParams(needs_layout_passes=False),
    scratch_types=[
        pltpu.VMEM((RPW,), jnp.int32),
        pltpu.VMEM((NBUF, CHUNK, EMBED), jnp.float32),
        [pltpu.SemaphoreType.DMA] * NBUF,
    ],
)
def _sc_gather(ids_hbm, table_hbm, out_hbm, idx_v, bufs, sems):
    wid = lax.axis_index("s") * NC + lax.axis_index("c")
    base = wid * RPW
    pltpu.sync_copy(ids_hbm.at[pl.ds(base, RPW)], idx_v)

    def start(k, b):
        pltpu.async_copy(
            table_hbm.at[idx_v.at[pl.ds(k * CHUNK, CHUNK)]], bufs.at[b], sems[b]
        )

    def wait(k, b):
        pltpu.make_async_copy(
            table_hbm.at[idx_v.at[pl.ds(k * CHUNK, CHUNK)]], bufs.at[b], sems[b]
        ).wait()

    for b in range(NBUF):
        start(b, b)

    def ring_body(step, _):
        for b in range(NBUF):
            k = step * NBUF + b
            wait(k, b)
            pltpu.sync_copy(bufs.at[b], out_hbm.at[pl.ds(base + k * CHUNK, CHUNK)])

            @pl.when(step < NCHUNK // NBUF - 1)
            def _():
                start(k + NBUF, b)

        return 0

    lax.fori_loop(0, NCHUNK // NBUF, ring_body, 0)


BM = 1024


def _tc_ln_body(emb_ref, pos_ref, g_ref, b_ref, out_ref):
    e = emb_ref[...] + pos_ref[...]
    mean = jnp.mean(e, axis=-1, keepdims=True)
    var = jnp.mean(e * e, axis=-1, keepdims=True) - mean * mean
    out_ref[...] = (e - mean) * lax.rsqrt(var + 1e-5) * g_ref[...] + b_ref[...]


_tc_ln = pl.pallas_call(
    _tc_ln_body,
    grid=(SEQ // BM, BATCH),
    in_specs=[
        pl.BlockSpec((BM, EMBED), lambda i, j: (j * (SEQ // BM) + i, 0)),
        pl.BlockSpec((BM, EMBED), lambda i, j: (i, 0)),
        pl.BlockSpec((1, EMBED), lambda i, j: (0, 0)),
        pl.BlockSpec((1, EMBED), lambda i, j: (0, 0)),
    ],
    out_specs=pl.BlockSpec((BM, EMBED), lambda i, j: (j * (SEQ // BM) + i, 0)),
    out_shape=jax.ShapeDtypeStruct((ROWS, EMBED), jnp.float32),
)


def kernel(input_ids, token_table, pos_table, gamma, beta):
    flat_ids = input_ids.reshape(-1).astype(jnp.int32)
    emb = _sc_gather(flat_ids, token_table)
    out = _tc_ln(emb, pos_table, gamma.reshape(1, EMBED), beta.reshape(1, EMBED))
    return out.reshape(BATCH, SEQ, EMBED)


# speedup vs baseline: 1.1742x; 1.0015x over previous
import functools

import jax
import jax.numpy as jnp
from jax import lax
from jax.experimental import pallas as pl
from jax.experimental.pallas import tpu as pltpu
from jax.experimental.pallas import tpu_sc as plsc

VOCAB = 100000
SEQ = 2048
BATCH = 4
EMBED = 1024

NC = 2
NS = 16
NW = NC * NS

ROWS = BATCH * SEQ
RPW = ROWS // NW
CHUNK = 16
NCHUNK = RPW // CHUNK
NBUF = 4

_mesh = plsc.VectorSubcoreMesh(
    core_axis_name="c", subcore_axis_name="s", num_cores=NC, num_subcores=NS
)


@functools.partial(
    pl.kernel,
    out_type=jax.ShapeDtypeStruct((ROWS, EMBED), jnp.float32),
    mesh=_mesh,
    compiler_params=pltpu.CompilerParams(needs_layout_passes=False),
    scratch_types=[
        pltpu.VMEM((RPW,), jnp.int32),
        pltpu.VMEM((NBUF, CHUNK, EMBED), jnp.float32),
        [pltpu.SemaphoreType.DMA] * NBUF,
    ],
)
def _sc_gather(ids_hbm, table_hbm, out_hbm, idx_v, bufs, sems):
    wid = lax.axis_index("s") * NC + lax.axis_index("c")
    base = wid * RPW
    pltpu.sync_copy(ids_hbm.at[pl.ds(base, RPW)], idx_v)

    def start(k, b):
        pltpu.async_copy(
            table_hbm.at[idx_v.at[pl.ds(k * CHUNK, CHUNK)]], bufs.at[b], sems[b]
        )

    def wait(k, b):
        pltpu.make_async_copy(
            table_hbm.at[idx_v.at[pl.ds(k * CHUNK, CHUNK)]], bufs.at[b], sems[b]
        ).wait()

    for b in range(NBUF):
        start(b, b)

    def ring_body(step, _):
        for b in range(NBUF):
            k = step * NBUF + b
            wait(k, b)
            pltpu.sync_copy(bufs.at[b], out_hbm.at[pl.ds(base + k * CHUNK, CHUNK)])

            @pl.when(step < NCHUNK // NBUF - 1)
            def _():
                start(k + NBUF, b)

        return 0

    lax.fori_loop(0, NCHUNK // NBUF, ring_body, 0)


BM = 2048


def _tc_ln_body(emb_ref, pos_ref, g_ref, b_ref, out_ref):
    e = emb_ref[...] + pos_ref[...]
    mean = jnp.mean(e, axis=-1, keepdims=True)
    var = jnp.mean(e * e, axis=-1, keepdims=True) - mean * mean
    out_ref[...] = (e - mean) * lax.rsqrt(var + 1e-5) * g_ref[...] + b_ref[...]


_tc_ln = pl.pallas_call(
    _tc_ln_body,
    grid=(SEQ // BM, BATCH),
    in_specs=[
        pl.BlockSpec((BM, EMBED), lambda i, j: (j * (SEQ // BM) + i, 0)),
        pl.BlockSpec((BM, EMBED), lambda i, j: (i, 0)),
        pl.BlockSpec((1, EMBED), lambda i, j: (0, 0)),
        pl.BlockSpec((1, EMBED), lambda i, j: (0, 0)),
    ],
    out_specs=pl.BlockSpec((BM, EMBED), lambda i, j: (j * (SEQ // BM) + i, 0)),
    out_shape=jax.ShapeDtypeStruct((ROWS, EMBED), jnp.float32),
)


def kernel(input_ids, token_table, pos_table, gamma, beta):
    flat_ids = input_ids.reshape(-1).astype(jnp.int32)
    emb = _sc_gather(flat_ids, token_table)
    out = _tc_ln(emb, pos_table, gamma.reshape(1, EMBED), beta.reshape(1, EMBED))
    return out.reshape(BATCH, SEQ, EMBED)
